# baseline (speedup 1.0000x reference)
import functools

import jax
import jax.numpy as jnp
from jax import lax
from jax.experimental import pallas as pl
from jax.experimental.pallas import tpu as pltpu
from jax.experimental.pallas import tpu_sc as plsc

_LANES = 128


def _sc_gather_fn(BATCH, SEQ, V, D, NC, NS, BR):
    NW = NC * NS
    rows_per_w = BATCH // NW
    n_chunks = rows_per_w // BR
    mesh = plsc.VectorSubcoreMesh(core_axis_name="c", subcore_axis_name="s")

    @functools.partial(
        pl.kernel,
        out_type=jax.ShapeDtypeStruct((BATCH, SEQ, _LANES), jnp.float32),
        mesh=mesh,
        compiler_params=pltpu.CompilerParams(use_tc_tiling_on_sc=False),
        scratch_types=[
            pltpu.VMEM((BR, SEQ), jnp.int32),
            pltpu.VMEM((BR, SEQ), jnp.int32),
            pltpu.VMEM((BR, SEQ, D), jnp.float32),
            pltpu.VMEM((BR, SEQ, D), jnp.float32),
            pltpu.SemaphoreType.DMA,
            pltpu.SemaphoreType.DMA,
            pltpu.SemaphoreType.DMA,
            pltpu.SemaphoreType.DMA,
            pltpu.SemaphoreType.DMA,
            pltpu.SemaphoreType.DMA,
        ],
    )
    def k(idx_hbm, table_hbm, out_hbm, idx0, idx1, rows0, rows1,
          g0sem, g1sem, s0sem, s1sem, i0sem, i1sem):
        wid = lax.axis_index("s") * NC + lax.axis_index("c")
        base = wid * rows_per_w
        idx_bufs = (idx0, idx1)
        row_bufs = (rows0, rows1)
        gsems = (g0sem, g1sem)
        ssems = (s0sem, s1sem)
        isems = (i0sem, i1sem)

        def idx_slice(chunk):
            return idx_hbm.at[pl.ds(base + chunk * BR, BR)]

        def out_slice(chunk):
            return out_hbm.at[pl.ds(base + chunk * BR, BR), :, pl.ds(0, D)]

        def start_gathers(b):
            for r in range(BR):
                pltpu.async_copy(
                    table_hbm.at[idx_bufs[b].at[r]], row_bufs[b].at[r],
                    gsems[b])

        def wait_gathers(b):
            for r in range(BR):
                pltpu.make_async_copy(
                    table_hbm.at[idx_bufs[b].at[r]], row_bufs[b].at[r],
                    gsems[b]).wait()

        pltpu.sync_copy(idx_slice(0), idx0)
        start_gathers(0)
        pltpu.async_copy(idx_slice(1), idx1, i1sem)

        @pl.loop(0, n_chunks, step=2)
        def _outer(g):
            for b in range(2):
                o = 1 - b
                chunk = g + b

                @pl.when(chunk + 1 < n_chunks)
                def _():
                    pltpu.make_async_copy(
                        idx_slice(chunk + 1), idx_bufs[o], isems[o]).wait()

                    @pl.when(chunk >= 1)
                    def _():
                        pltpu.make_async_copy(
                            row_bufs[o], out_slice(chunk - 1),
                            ssems[o]).wait()

                    start_gathers(o)

                wait_gathers(b)
                pltpu.async_copy(row_bufs[b], out_slice(chunk), ssems[b])

                @pl.when(chunk + 2 < n_chunks)
                def _():
                    pltpu.async_copy(
                        idx_slice(chunk + 2), idx_bufs[b], isems[b])

        last = n_chunks - 1
        pltpu.make_async_copy(
            row_bufs[last % 2], out_slice(last), ssems[last % 2]).wait()

    return k


def _tc_transpose_seg_fn(BATCH, SEG, SEQ, D, BB, SP, seg, first):
    n_s = SEQ // SP
    n_b = SEG // BB
    seg_blocks = seg * (SEG // BB)

    def body(*refs):
        x_ref, ot_ref = refs[-2], refs[-1]
        x = x_ref[...]
        x2 = x.reshape(BB, SP * _LANES)
        y = jnp.swapaxes(x2, 0, 1)
        y3 = y.reshape(SP, _LANES, BB)
        ot_ref[...] = y3[:, :D, :]

    x_spec = pl.BlockSpec((BB, SP, _LANES), lambda sp, bb: (bb, sp, 0))
    out_spec = pl.BlockSpec(
        (SP, D, BB), lambda sp, bb: (sp, 0, seg_blocks + bb))
    out_shape = jax.ShapeDtypeStruct((SEQ, D, BATCH), jnp.float32)
    if first:
        return pl.pallas_call(
            body, grid=(n_s, n_b), in_specs=[x_spec],
            out_specs=out_spec, out_shape=out_shape)
    return pl.pallas_call(
        body, grid=(n_s, n_b),
        in_specs=[pl.BlockSpec(memory_space=pl.ANY), x_spec],
        out_specs=out_spec, out_shape=out_shape,
        input_output_aliases={0: 0})


def kernel(position_ids, position_embeddings):
    batch, seq = position_ids.shape
    V, D = position_embeddings.shape
    K = 4
    seg_rows = batch // K
    fn = _sc_gather_fn(seg_rows, seq, V, D, 2, 16, 4)
    xs = [
        fn(position_ids[k * seg_rows:(k + 1) * seg_rows],
           position_embeddings)
        for k in range(K)
    ]
    ot = _tc_transpose_seg_fn(
        batch, seg_rows, seq, D, 1024, 8, 0, True)(xs[0])
    for k in range(1, K):
        ot = _tc_transpose_seg_fn(
            batch, seg_rows, seq, D, 1024, 8, k, False)(ot, xs[k])
    return jnp.transpose(ot, (2, 0, 1))


# speedup vs baseline: 1.0086x; 1.0086x over previous
import functools

import jax
import jax.numpy as jnp
from jax import lax
from jax.experimental import pallas as pl
from jax.experimental.pallas import tpu as pltpu
from jax.experimental.pallas import tpu_sc as plsc

_LANES = 128


def _sc_gather_fn(BATCH, SEQ, V, D, NC, NS, BR):
    NW = NC * NS
    rows_per_w = BATCH // NW
    n_chunks = rows_per_w // BR
    mesh = plsc.VectorSubcoreMesh(core_axis_name="c", subcore_axis_name="s")

    @functools.partial(
        pl.kernel,
        out_type=jax.ShapeDtypeStruct((BATCH, SEQ, _LANES), jnp.float32),
        mesh=mesh,
        compiler_params=pltpu.CompilerParams(use_tc_tiling_on_sc=False),
        scratch_types=[
            pltpu.VMEM((BR, SEQ), jnp.int32),
            pltpu.VMEM((BR, SEQ), jnp.int32),
            pltpu.VMEM((BR, SEQ, D), jnp.float32),
            pltpu.VMEM((BR, SEQ, D), jnp.float32),
            pltpu.SemaphoreType.DMA,
            pltpu.SemaphoreType.DMA,
            pltpu.SemaphoreType.DMA,
            pltpu.SemaphoreType.DMA,
            pltpu.SemaphoreType.DMA,
            pltpu.SemaphoreType.DMA,
        ],
    )
    def k(idx_hbm, table_hbm, out_hbm, idx0, idx1, rows0, rows1,
          g0sem, g1sem, s0sem, s1sem, i0sem, i1sem):
        wid = lax.axis_index("s") * NC + lax.axis_index("c")
        base = wid * rows_per_w
        idx_bufs = (idx0, idx1)
        row_bufs = (rows0, rows1)
        gsems = (g0sem, g1sem)
        ssems = (s0sem, s1sem)
        isems = (i0sem, i1sem)

        def idx_slice(chunk):
            return idx_hbm.at[pl.ds(base + chunk * BR, BR)]

        def out_slice(chunk):
            return out_hbm.at[pl.ds(base + chunk * BR, BR), :, pl.ds(0, D)]

        def start_gathers(b):
            for r in range(BR):
                pltpu.async_copy(
                    table_hbm.at[idx_bufs[b].at[r]], row_bufs[b].at[r],
                    gsems[b])

        def wait_gathers(b):
            for r in range(BR):
                pltpu.make_async_copy(
                    table_hbm.at[idx_bufs[b].at[r]], row_bufs[b].at[r],
                    gsems[b]).wait()

        pltpu.sync_copy(idx_slice(0), idx0)
        start_gathers(0)
        pltpu.async_copy(idx_slice(1), idx1, i1sem)

        @pl.loop(0, n_chunks, step=2)
        def _outer(g):
            for b in range(2):
                o = 1 - b
                chunk = g + b

                @pl.when(chunk + 1 < n_chunks)
                def _():
                    pltpu.make_async_copy(
                        idx_slice(chunk + 1), idx_bufs[o], isems[o]).wait()

                    @pl.when(chunk >= 1)
                    def _():
                        pltpu.make_async_copy(
                            row_bufs[o], out_slice(chunk - 1),
                            ssems[o]).wait()

                    start_gathers(o)

                wait_gathers(b)
                pltpu.async_copy(row_bufs[b], out_slice(chunk), ssems[b])

                @pl.when(chunk + 2 < n_chunks)
                def _():
                    pltpu.async_copy(
                        idx_slice(chunk + 2), idx_bufs[b], isems[b])

        last = n_chunks - 1
        pltpu.make_async_copy(
            row_bufs[last % 2], out_slice(last), ssems[last % 2]).wait()

    return k


def _tc_transpose_seg_fn(BATCH, SEG, SEQ, D, BB, SP, seg, first):
    n_s = SEQ // SP
    n_b = SEG // BB
    seg_blocks = seg * (SEG // BB)

    def body(*refs):
        x_ref, ot_ref = refs[-2], refs[-1]
        x = x_ref[...]
        x2 = x.reshape(BB, SP * _LANES)
        y = jnp.swapaxes(x2, 0, 1)
        y3 = y.reshape(SP, _LANES, BB)
        ot_ref[...] = y3[:, :D, :]

    x_spec = pl.BlockSpec((BB, SP, _LANES), lambda sp, bb: (bb, sp, 0))
    out_spec = pl.BlockSpec(
        (SP, D, BB), lambda sp, bb: (sp, 0, seg_blocks + bb))
    out_shape = jax.ShapeDtypeStruct((SEQ, D, BATCH), jnp.float32)
    if first:
        return pl.pallas_call(
            body, grid=(n_s, n_b), in_specs=[x_spec],
            out_specs=out_spec, out_shape=out_shape)
    return pl.pallas_call(
        body, grid=(n_s, n_b),
        in_specs=[pl.BlockSpec(memory_space=pl.ANY), x_spec],
        out_specs=out_spec, out_shape=out_shape,
        input_output_aliases={0: 0})


def kernel(position_ids, position_embeddings):
    batch, seq = position_ids.shape
    V, D = position_embeddings.shape
    K = 2
    seg_rows = batch // K
    fn = _sc_gather_fn(seg_rows, seq, V, D, 2, 16, 4)
    xs = [
        fn(position_ids[k * seg_rows:(k + 1) * seg_rows],
           position_embeddings)
        for k in range(K)
    ]
    ot = _tc_transpose_seg_fn(
        batch, seg_rows, seq, D, 2048, 8, 0, True)(xs[0])
    for k in range(1, K):
        ot = _tc_transpose_seg_fn(
            batch, seg_rows, seq, D, 2048, 8, k, False)(ot, xs[k])
    return jnp.transpose(ot, (2, 0, 1))


# speedup vs baseline: 1.0204x; 1.0116x over previous
import functools

import jax
import jax.numpy as jnp
from jax import lax
from jax.experimental import pallas as pl
from jax.experimental.pallas import tpu as pltpu
from jax.experimental.pallas import tpu_sc as plsc

_LANES = 128


def _sc_gather_fn(BATCH, SEQ, V, D, NC, NS, BR):
    NW = NC * NS
    rows_per_w = BATCH // NW
    n_chunks = rows_per_w // BR
    mesh = plsc.VectorSubcoreMesh(core_axis_name="c", subcore_axis_name="s")

    @functools.partial(
        pl.kernel,
        out_type=jax.ShapeDtypeStruct((BATCH, SEQ, _LANES), jnp.float32),
        mesh=mesh,
        compiler_params=pltpu.CompilerParams(use_tc_tiling_on_sc=False),
        scratch_types=[
            pltpu.VMEM((BR, SEQ), jnp.int32),
            pltpu.VMEM((BR, SEQ), jnp.int32),
            pltpu.VMEM((BR, SEQ, D), jnp.float32),
            pltpu.VMEM((BR, SEQ, D), jnp.float32),
            pltpu.SemaphoreType.DMA,
            pltpu.SemaphoreType.DMA,
            pltpu.SemaphoreType.DMA,
            pltpu.SemaphoreType.DMA,
            pltpu.SemaphoreType.DMA,
            pltpu.SemaphoreType.DMA,
        ],
    )
    def k(idx_hbm, table_hbm, out_hbm, idx0, idx1, rows0, rows1,
          g0sem, g1sem, s0sem, s1sem, i0sem, i1sem):
        wid = lax.axis_index("s") * NC + lax.axis_index("c")
        base = wid * rows_per_w
        idx_bufs = (idx0, idx1)
        row_bufs = (rows0, rows1)
        gsems = (g0sem, g1sem)
        ssems = (s0sem, s1sem)
        isems = (i0sem, i1sem)

        def idx_slice(chunk):
            return idx_hbm.at[pl.ds(base + chunk * BR, BR)]

        def out_slice(chunk):
            return out_hbm.at[pl.ds(base + chunk * BR, BR), :, pl.ds(0, D)]

        def start_gathers(b):
            for r in range(BR):
                pltpu.async_copy(
                    table_hbm.at[idx_bufs[b].at[r]], row_bufs[b].at[r],
                    gsems[b])

        def wait_gathers(b):
            for r in range(BR):
                pltpu.make_async_copy(
                    table_hbm.at[idx_bufs[b].at[r]], row_bufs[b].at[r],
                    gsems[b]).wait()

        pltpu.sync_copy(idx_slice(0), idx0)
        start_gathers(0)
        pltpu.async_copy(idx_slice(1), idx1, i1sem)

        @pl.loop(0, n_chunks, step=2)
        def _outer(g):
            for b in range(2):
                o = 1 - b
                chunk = g + b

                @pl.when(chunk + 1 < n_chunks)
                def _():
                    pltpu.make_async_copy(
                        idx_slice(chunk + 1), idx_bufs[o], isems[o]).wait()

                    @pl.when(chunk >= 1)
                    def _():
                        pltpu.make_async_copy(
                            row_bufs[o], out_slice(chunk - 1),
                            ssems[o]).wait()

                    start_gathers(o)

                wait_gathers(b)
                pltpu.async_copy(row_bufs[b], out_slice(chunk), ssems[b])

                @pl.when(chunk + 2 < n_chunks)
                def _():
                    pltpu.async_copy(
                        idx_slice(chunk + 2), idx_bufs[b], isems[b])

        for last in (n_chunks - 2, n_chunks - 1):
            pltpu.make_async_copy(
                row_bufs[last % 2], out_slice(last), ssems[last % 2]).wait()

    return k


def _tc_transpose_seg_fn(BATCH, SEG, SEQ, D, BB, SP, seg, first):
    n_s = SEQ // SP
    n_b = SEG // BB
    seg_blocks = seg * (SEG // BB)

    def body(*refs):
        x_ref, ot_ref = refs[-2], refs[-1]
        x = x_ref[...]
        x2 = x.reshape(BB, SP * _LANES)
        y = jnp.swapaxes(x2, 0, 1)
        y3 = y.reshape(SP, _LANES, BB)
        ot_ref[...] = y3[:, :D, :]

    x_spec = pl.BlockSpec((BB, SP, _LANES), lambda sp, bb: (bb, sp, 0))
    out_spec = pl.BlockSpec(
        (SP, D, BB), lambda sp, bb: (sp, 0, seg_blocks + bb))
    out_shape = jax.ShapeDtypeStruct((SEQ, D, BATCH), jnp.float32)
    if first:
        return pl.pallas_call(
            body, grid=(n_s, n_b), in_specs=[x_spec],
            out_specs=out_spec, out_shape=out_shape)
    return pl.pallas_call(
        body, grid=(n_s, n_b),
        in_specs=[pl.BlockSpec(memory_space=pl.ANY), x_spec],
        out_specs=out_spec, out_shape=out_shape,
        input_output_aliases={0: 0})


def kernel(position_ids, position_embeddings):
    batch, seq = position_ids.shape
    V, D = position_embeddings.shape
    fn = _sc_gather_fn(batch, seq, V, D, 2, 16, 4)
    xpad = fn(position_ids, position_embeddings)
    ot = _tc_transpose_seg_fn(
        batch, batch, seq, D, 4096, 8, 0, True)(xpad)
    return jnp.transpose(ot, (2, 0, 1))
